# baseline (speedup 1.0000x reference)
import functools

import jax
import jax.numpy as jnp
from jax import lax
from jax.experimental import pallas as pl
from jax.experimental.pallas import tpu as pltpu
from jax.experimental.pallas import tpu_sc as plsc

N = 10000
E = 320000
D = 128
LANES = 16
NC = 2
NS = 16
NW = NC * NS
CHUNK = 128
NCHUNK = 2560
EPAD = NCHUNK * CHUNK
CPW = NCHUNK // NW
IBLK = 8
NBLK = CPW // IBLK
N_PAD = 10240
RPW = N_PAD // NS
ZROWS = 8


def _sc_body(x_hbm, ei_hbm, s0_hbm, s1_hbm, d0_hbm, d1_hbm,
             idx_s, idx_d, rows0, rows1, ones, zbuf, zdeg, s_sh, deg_sh,
             sem_g0, sem_g1, sem_s0, sem_s1, sem_o0, sem_o1):
    c = lax.axis_index("c")
    s = lax.axis_index("s")
    wid = c * NS + s
    rows = (rows0, rows1)
    sem_g = (sem_g0, sem_g1)
    sem_s = (sem_s0, sem_s1)
    sem_o = (sem_o0, sem_o1)

    def zbuf_body(i, _):
        zbuf[i // 8, pl.ds((i % 8) * LANES, LANES)] = jnp.zeros((LANES,), jnp.float32)
        return 0
    lax.fori_loop(0, ZROWS * 8, zbuf_body, 0)

    def zdeg_body(i, _):
        zdeg[i] = jnp.zeros((LANES,), jnp.float32)
        return 0
    lax.fori_loop(0, 16, zdeg_body, 0)

    def ones_body(i, _):
        ones[i] = jnp.ones((LANES,), jnp.float32)
        return 0
    lax.fori_loop(0, CHUNK, ones_body, 0)

    def z_s(k, _):
        pltpu.sync_copy(zbuf, s_sh.at[pl.ds(s * RPW + k * ZROWS, ZROWS)])
        return 0
    lax.fori_loop(0, RPW // ZROWS, z_s, 0)

    def z_d(k, _):
        pltpu.sync_copy(zdeg, deg_sh.at[pl.ds(s * RPW + k * 16, 16)])
        return 0
    lax.fori_loop(0, RPW // 16, z_d, 0)
    plsc.subcore_barrier()

    base = wid * CPW

    def block_body(b, _):
        pltpu.sync_copy(ei_hbm.at[0, pl.ds(base + b * IBLK, IBLK)], idx_s)
        pltpu.sync_copy(ei_hbm.at[1, pl.ds(base + b * IBLK, IBLK)], idx_d)
        g = [None, None]
        sc = [None, None]
        oc = [None, None]
        g[0] = pltpu.async_copy(x_hbm.at[idx_s.at[0]], rows0, sem_g0)
        for j in range(IBLK):
            p = j & 1
            q = 1 - p
            g[p].wait()
            if sc[q] is not None:
                sc[q].wait()
                oc[q].wait()
            if j + 1 < IBLK:
                g[q] = pltpu.async_copy(
                    x_hbm.at[idx_s.at[j + 1]], rows[q], sem_g[q])
            sc[p] = pltpu.async_copy(
                rows[p], s_sh.at[idx_d.at[j]], sem_s[p], add=True)
            oc[p] = pltpu.async_copy(
                ones, deg_sh.at[idx_d.at[j]], sem_o[p], add=True)
        last = (IBLK - 1) & 1
        sc[last].wait()
        oc[last].wait()
        return 0
    lax.fori_loop(0, NBLK, block_body, 0)
    plsc.subcore_barrier()

    @pl.when(c == 0)
    def _():
        pltpu.sync_copy(s_sh.at[pl.ds(s * RPW, RPW)],
                        s0_hbm.at[pl.ds(s * RPW, RPW)])
        pltpu.sync_copy(deg_sh.at[pl.ds(s * RPW, RPW)],
                        d0_hbm.at[pl.ds(s * RPW, RPW)])

    @pl.when(c == 1)
    def _():
        pltpu.sync_copy(s_sh.at[pl.ds(s * RPW, RPW)],
                        s1_hbm.at[pl.ds(s * RPW, RPW)])
        pltpu.sync_copy(deg_sh.at[pl.ds(s * RPW, RPW)],
                        d1_hbm.at[pl.ds(s * RPW, RPW)])


@functools.lru_cache(maxsize=1)
def _make_sc_segsum():
  return functools.partial(
    pl.kernel,
    out_type=(
        jax.ShapeDtypeStruct((N_PAD, D), jnp.float32),
        jax.ShapeDtypeStruct((N_PAD, D), jnp.float32),
        jax.ShapeDtypeStruct((N_PAD, LANES), jnp.float32),
        jax.ShapeDtypeStruct((N_PAD, LANES), jnp.float32),
    ),
    mesh=plsc.VectorSubcoreMesh(core_axis_name="c", subcore_axis_name="s",
                                num_cores=NC, num_subcores=NS),
    scratch_types=[
        pltpu.VMEM((IBLK, CHUNK), jnp.int32),
        pltpu.VMEM((IBLK, CHUNK), jnp.int32),
        pltpu.VMEM((CHUNK, D), jnp.float32),
        pltpu.VMEM((CHUNK, D), jnp.float32),
        pltpu.VMEM((CHUNK, LANES), jnp.float32),
        pltpu.VMEM((ZROWS, D), jnp.float32),
        pltpu.VMEM((16, LANES), jnp.float32),
        pltpu.VMEM_SHARED((N_PAD, D), jnp.float32),
        pltpu.VMEM_SHARED((N_PAD, LANES), jnp.float32),
        pltpu.SemaphoreType.DMA,
        pltpu.SemaphoreType.DMA,
        pltpu.SemaphoreType.DMA,
        pltpu.SemaphoreType.DMA,
        pltpu.SemaphoreType.DMA,
        pltpu.SemaphoreType.DMA,
    ],
    compiler_params=pltpu.CompilerParams(use_tc_tiling_on_sc=False),
  )(_sc_body)


def _tc_a_body(x_ref, we_ref, be_ref, wn_ref, bn_ref, z_ref, m1t_ref):
    w1 = we_ref[:, :D]
    w2 = we_ref[:, D:]
    wnt = wn_ref[...].T
    zx = jnp.dot(x_ref[...], w2.T, preferred_element_type=jnp.float32) + be_ref[...]
    z_ref[...] = jnp.dot(zx, wnt, preferred_element_type=jnp.float32) + bn_ref[...]
    m1t_ref[...] = jnp.dot(w1.T, wnt, preferred_element_type=jnp.float32)


def _tc_a(x, w_edge, b_edge, w_node, b_node):
    blk = 1000
    row_spec = pl.BlockSpec((blk, D), lambda i: (i, 0))
    full = lambda a, b: pl.BlockSpec((a, b), lambda i: (0, 0))
    return pl.pallas_call(
        _tc_a_body,
        grid=(N // blk,),
        in_specs=[row_spec, full(D, 2 * D), full(1, D), full(D, D), full(1, D)],
        out_specs=[row_spec, full(D, D)],
        out_shape=[
            jax.ShapeDtypeStruct((N, D), jnp.float32),
            jax.ShapeDtypeStruct((D, D), jnp.float32),
        ],
    )(x, w_edge, b_edge, w_node, b_node)


def _tc_b_body(x_ref, s0_ref, s1_ref, d0_hbm, d1_hbm, z_ref, m1t_ref,
               out_ref, dv0, dv1, sem0, sem1):
    blk = out_ref.shape[0]
    i = pl.program_id(0)
    c0 = pltpu.make_async_copy(d0_hbm.at[pl.ds(i * blk, blk)], dv0, sem0)
    c1 = pltpu.make_async_copy(d1_hbm.at[pl.ds(i * blk, blk)], dv1, sem1)
    c0.start()
    c1.start()
    c0.wait()
    c1.wait()
    deg = dv0[:, 0:1] + dv1[:, 0:1]
    inv = 1.0 / jnp.maximum(deg, 1.0)
    mean_s = (s0_ref[...] + s1_ref[...]) * inv
    h = jnp.dot(mean_s, m1t_ref[...], preferred_element_type=jnp.float32) + z_ref[...]
    out_ref[...] = jnp.where(deg > 0.0, h, x_ref[...])


def _tc_b(x, s0, s1, d0, d1, z, m1t):
    blk = 1000
    row_spec = pl.BlockSpec((blk, D), lambda i: (i, 0))
    any_spec = pl.BlockSpec(memory_space=pltpu.MemorySpace.HBM)
    full = lambda a, b: pl.BlockSpec((a, b), lambda i: (0, 0))
    return pl.pallas_call(
        _tc_b_body,
        grid=(N // blk,),
        in_specs=[row_spec, row_spec, row_spec, any_spec, any_spec,
                  row_spec, full(D, D)],
        out_specs=row_spec,
        out_shape=jax.ShapeDtypeStruct((N, D), jnp.float32),
        scratch_shapes=[
            pltpu.VMEM((blk, LANES), jnp.float32),
            pltpu.VMEM((blk, LANES), jnp.float32),
            pltpu.SemaphoreType.DMA,
            pltpu.SemaphoreType.DMA,
        ],
    )(x, s0, s1, d0, d1, z, m1t)


def kernel(node_inputs, edge_index, W_edge, b_edge, W_node, b_node):
    npad = EPAD - E
    src_p = jnp.concatenate([edge_index[0], jnp.zeros((npad,), jnp.int32)])
    fakes = N + jnp.arange(npad, dtype=jnp.int32) % (N_PAD - N)
    dst_p = jnp.concatenate([edge_index[1], fakes])
    ei = jnp.stack([src_p, dst_p]).reshape(2, NCHUNK, CHUNK)
    s0, s1, d0, d1 = _make_sc_segsum()(node_inputs, ei)
    z, m1t = _tc_a(node_inputs, W_edge, b_edge.reshape(1, D),
                   W_node, b_node.reshape(1, D))
    return _tc_b(node_inputs, s0, s1, d0, d1, z, m1t)


# speedup vs baseline: 2.9219x; 2.9219x over previous
import functools

import jax
import jax.numpy as jnp
from jax import lax
from jax.experimental import pallas as pl
from jax.experimental.pallas import tpu as pltpu
from jax.experimental.pallas import tpu_sc as plsc

N = 10000
E = 320000
D = 128
LANES = 16
NC = 2
NS = 16
NW = NC * NS
CHUNK = 128
NCHUNK = 2560
EPAD = NCHUNK * CHUNK
CPW = NCHUNK // NW
IBLK = 8
NBLK = CPW // IBLK
N_PAD = 10240
RPW = N_PAD // NS
ZROWS = 8


def _sc_body(x_hbm, ei_hbm, s0_hbm, s1_hbm, d0_hbm, d1_hbm,
             idx_s, idx_d, rows0, rows1, ones, zbuf, zdeg, s_sh, deg_sh,
             sem_g0, sem_g1, sem_s0, sem_s1, sem_o0, sem_o1):
    c = lax.axis_index("c")
    s = lax.axis_index("s")
    wid = c * NS + s
    rows = (rows0, rows1)
    sem_g = (sem_g0, sem_g1)
    sem_s = (sem_s0, sem_s1)
    sem_o = (sem_o0, sem_o1)

    def zbuf_body(i, _):
        zbuf[i // 8, pl.ds((i % 8) * LANES, LANES)] = jnp.zeros((LANES,), jnp.float32)
        return 0
    lax.fori_loop(0, ZROWS * 8, zbuf_body, 0)

    def zdeg_body(i, _):
        zdeg[i] = jnp.zeros((LANES,), jnp.float32)
        return 0
    lax.fori_loop(0, 16, zdeg_body, 0)

    def ones_body(i, _):
        ones[i] = jnp.ones((LANES,), jnp.float32)
        return 0
    lax.fori_loop(0, CHUNK, ones_body, 0)

    def z_s(k, _):
        pltpu.sync_copy(zbuf, s_sh.at[pl.ds(s * RPW + k * ZROWS, ZROWS)])
        return 0
    lax.fori_loop(0, RPW // ZROWS, z_s, 0)

    def z_d(k, _):
        pltpu.sync_copy(zdeg, deg_sh.at[pl.ds(s * RPW + k * 16, 16)])
        return 0
    lax.fori_loop(0, RPW // 16, z_d, 0)
    plsc.subcore_barrier()

    base = wid * CPW

    def block_body(b, _):
        pltpu.sync_copy(ei_hbm.at[0, pl.ds(base + b * IBLK, IBLK)], idx_s)
        pltpu.sync_copy(ei_hbm.at[1, pl.ds(base + b * IBLK, IBLK)], idx_d)
        g = [None, None]
        sc = [None, None]
        oc = [None, None]
        g[0] = pltpu.async_copy(x_hbm.at[idx_s.at[0]], rows0, sem_g0)
        for j in range(IBLK):
            p = j & 1
            q = 1 - p
            g[p].wait()
            if sc[q] is not None:
                sc[q].wait()
                oc[q].wait()
            if j + 1 < IBLK:
                g[q] = pltpu.async_copy(
                    x_hbm.at[idx_s.at[j + 1]], rows[q], sem_g[q])
            sc[p] = pltpu.async_copy(
                rows[p], s_sh.at[idx_d.at[j]], sem_s[p], add=True)
            oc[p] = pltpu.async_copy(
                ones, deg_sh.at[idx_d.at[j]], sem_o[p], add=True)
        last = (IBLK - 1) & 1
        sc[last].wait()
        oc[last].wait()
        return 0
    lax.fori_loop(0, NBLK, block_body, 0)
    plsc.subcore_barrier()

    @pl.when(c == 0)
    def _():
        pltpu.sync_copy(s_sh.at[pl.ds(s * RPW, RPW)],
                        s0_hbm.at[pl.ds(s * RPW, RPW)])
        pltpu.sync_copy(deg_sh.at[pl.ds(s * RPW, RPW)],
                        d0_hbm.at[pl.ds(s * RPW, RPW)])

    @pl.when(c == 1)
    def _():
        pltpu.sync_copy(s_sh.at[pl.ds(s * RPW, RPW)],
                        s1_hbm.at[pl.ds(s * RPW, RPW)])
        pltpu.sync_copy(deg_sh.at[pl.ds(s * RPW, RPW)],
                        d1_hbm.at[pl.ds(s * RPW, RPW)])


@functools.lru_cache(maxsize=1)
def _make_sc_segsum():
  return functools.partial(
    pl.kernel,
    out_type=(
        jax.ShapeDtypeStruct((N_PAD, D), jnp.float32),
        jax.ShapeDtypeStruct((N_PAD, D), jnp.float32),
        jax.ShapeDtypeStruct((N_PAD, LANES), jnp.float32),
        jax.ShapeDtypeStruct((N_PAD, LANES), jnp.float32),
    ),
    mesh=plsc.VectorSubcoreMesh(core_axis_name="c", subcore_axis_name="s",
                                num_cores=NC, num_subcores=NS),
    scratch_types=[
        pltpu.VMEM((IBLK, CHUNK), jnp.int32),
        pltpu.VMEM((IBLK, CHUNK), jnp.int32),
        pltpu.VMEM((CHUNK, D), jnp.float32),
        pltpu.VMEM((CHUNK, D), jnp.float32),
        pltpu.VMEM((CHUNK, LANES), jnp.float32),
        pltpu.VMEM((ZROWS, D), jnp.float32),
        pltpu.VMEM((16, LANES), jnp.float32),
        pltpu.VMEM_SHARED((N_PAD, D), jnp.float32),
        pltpu.VMEM_SHARED((N_PAD, LANES), jnp.float32),
        pltpu.SemaphoreType.DMA,
        pltpu.SemaphoreType.DMA,
        pltpu.SemaphoreType.DMA,
        pltpu.SemaphoreType.DMA,
        pltpu.SemaphoreType.DMA,
        pltpu.SemaphoreType.DMA,
    ],
    compiler_params=pltpu.CompilerParams(use_tc_tiling_on_sc=False),
  )(_sc_body)


def _tc_a_body(x_ref, we_ref, be_ref, wn_ref, bn_ref, z_ref, m1t_ref):
    w1 = we_ref[:, :D]
    w2 = we_ref[:, D:]
    wnt = wn_ref[...].T
    zx = jnp.dot(x_ref[...], w2.T, preferred_element_type=jnp.float32) + be_ref[...]
    z_ref[...] = jnp.dot(zx, wnt, preferred_element_type=jnp.float32) + bn_ref[...]
    m1t_ref[...] = jnp.dot(w1.T, wnt, preferred_element_type=jnp.float32)


def _tc_a(x, w_edge, b_edge, w_node, b_node):
    blk = 1000
    row_spec = pl.BlockSpec((blk, D), lambda i: (i, 0))
    full = lambda a, b: pl.BlockSpec((a, b), lambda i: (0, 0))
    return pl.pallas_call(
        _tc_a_body,
        grid=(N // blk,),
        in_specs=[row_spec, full(D, 2 * D), full(1, D), full(D, D), full(1, D)],
        out_specs=[row_spec, full(D, D)],
        out_shape=[
            jax.ShapeDtypeStruct((N, D), jnp.float32),
            jax.ShapeDtypeStruct((D, D), jnp.float32),
        ],
    )(x, w_edge, b_edge, w_node, b_node)


def _tc_b_body(x_ref, s0_ref, s1_ref, d0_hbm, d1_hbm, z_ref, m1t_ref,
               out_ref, dv0, dv1, sem0, sem1):
    blk = out_ref.shape[0]
    i = pl.program_id(0)
    c0 = pltpu.make_async_copy(d0_hbm.at[pl.ds(i * blk, blk)], dv0, sem0)
    c1 = pltpu.make_async_copy(d1_hbm.at[pl.ds(i * blk, blk)], dv1, sem1)
    c0.start()
    c1.start()
    c0.wait()
    c1.wait()
    deg = dv0[:, 0:1] + dv1[:, 0:1]
    inv = 1.0 / jnp.maximum(deg, 1.0)
    mean_s = (s0_ref[...] + s1_ref[...]) * inv
    h = jnp.dot(mean_s, m1t_ref[...], preferred_element_type=jnp.float32) + z_ref[...]
    out_ref[...] = jnp.where(deg > 0.0, h, x_ref[...])


def _tc_b(x, s0, s1, d0, d1, z, m1t):
    blk = 1000
    row_spec = pl.BlockSpec((blk, D), lambda i: (i, 0))
    any_spec = pl.BlockSpec(memory_space=pltpu.MemorySpace.HBM)
    full = lambda a, b: pl.BlockSpec((a, b), lambda i: (0, 0))
    return pl.pallas_call(
        _tc_b_body,
        grid=(N // blk,),
        in_specs=[row_spec, row_spec, row_spec, any_spec, any_spec,
                  row_spec, full(D, D)],
        out_specs=row_spec,
        out_shape=jax.ShapeDtypeStruct((N, D), jnp.float32),
        scratch_shapes=[
            pltpu.VMEM((blk, LANES), jnp.float32),
            pltpu.VMEM((blk, LANES), jnp.float32),
            pltpu.SemaphoreType.DMA,
            pltpu.SemaphoreType.DMA,
        ],
    )(x, s0, s1, d0, d1, z, m1t)


def kernel(node_inputs, edge_index, W_edge, b_edge, W_node, b_node):
    npad = EPAD - E
    fsrc = jnp.arange(npad, dtype=jnp.int32) * 13 % N
    src_p = jnp.concatenate([edge_index[0], fsrc])
    fakes = N + jnp.arange(npad, dtype=jnp.int32) % (N_PAD - N)
    dst_p = jnp.concatenate([edge_index[1], fakes])
    ei = jnp.stack([src_p, dst_p]).reshape(2, NCHUNK, CHUNK)
    s0, s1, d0, d1 = _make_sc_segsum()(node_inputs, ei)
    z, m1t = _tc_a(node_inputs, W_edge, b_edge.reshape(1, D),
                   W_node, b_node.reshape(1, D))
    return _tc_b(node_inputs, s0, s1, d0, d1, z, m1t)


# speedup vs baseline: 3.0435x; 1.0416x over previous
import functools

import jax
import jax.numpy as jnp
from jax import lax
from jax.experimental import pallas as pl
from jax.experimental.pallas import tpu as pltpu
from jax.experimental.pallas import tpu_sc as plsc

N = 10000
E = 320000
D = 128
LANES = 16
NC = 2
NS = 16
NW = NC * NS
CHUNK = 128
NCHUNK = 2560
EPAD = NCHUNK * CHUNK
CPW = NCHUNK // NW
IBLK = 8
NBLK = CPW // IBLK
N_PAD = 10240
RPW = N_PAD // NS
ZROWS = 8


def _sc_body(x_hbm, ei_hbm, s0_hbm, s1_hbm, d0_hbm, d1_hbm,
             idx_s, idx_d, rows0, rows1, ones, zbuf, zdeg, s_sh, deg_sh,
             sem_g0, sem_g1, sem_s0, sem_s1, sem_o0, sem_o1):
    c = lax.axis_index("c")
    s = lax.axis_index("s")
    wid = c * NS + s
    rows = (rows0, rows1)
    sem_g = (sem_g0, sem_g1)
    sem_s = (sem_s0, sem_s1)
    sem_o = (sem_o0, sem_o1)

    def zbuf_body(i, _):
        zbuf[i // 8, pl.ds((i % 8) * LANES, LANES)] = jnp.zeros((LANES,), jnp.float32)
        return 0
    lax.fori_loop(0, ZROWS * 8, zbuf_body, 0)

    def zdeg_body(i, _):
        zdeg[i] = jnp.zeros((LANES,), jnp.float32)
        return 0
    lax.fori_loop(0, 16, zdeg_body, 0)

    def ones_body(i, _):
        ones[i] = jnp.ones((LANES,), jnp.float32)
        return 0
    lax.fori_loop(0, CHUNK, ones_body, 0)

    def z_s(k, _):
        pltpu.sync_copy(zbuf, s_sh.at[pl.ds(s * RPW + k * ZROWS, ZROWS)])
        return 0
    lax.fori_loop(0, RPW // ZROWS, z_s, 0)

    def z_d(k, _):
        pltpu.sync_copy(zdeg, deg_sh.at[pl.ds(s * RPW + k * 16, 16)])
        return 0
    lax.fori_loop(0, RPW // 16, z_d, 0)
    plsc.subcore_barrier()

    base = wid * CPW

    def block_body(b, _):
        pltpu.sync_copy(ei_hbm.at[0, pl.ds(base + b * IBLK, IBLK)], idx_s)
        pltpu.sync_copy(ei_hbm.at[1, pl.ds(base + b * IBLK, IBLK)], idx_d)
        g = [None, None]
        sc = [None, None]
        oc = [None, None]
        g[0] = pltpu.async_copy(x_hbm.at[idx_s.at[0]], rows0, sem_g0)
        for j in range(IBLK):
            p = j & 1
            q = 1 - p
            g[p].wait()
            if sc[q] is not None:
                sc[q].wait()
                oc[q].wait()
            if j + 1 < IBLK:
                g[q] = pltpu.async_copy(
                    x_hbm.at[idx_s.at[j + 1]], rows[q], sem_g[q])
            sc[p] = pltpu.async_copy(
                rows[p], s_sh.at[idx_d.at[j]], sem_s[p], add=True)
            oc[p] = pltpu.async_copy(
                ones, deg_sh.at[idx_d.at[j]], sem_o[p], add=True)
        last = (IBLK - 1) & 1
        sc[last].wait()
        oc[last].wait()
        return 0
    lax.fori_loop(0, NBLK, block_body, 0)
    plsc.subcore_barrier()

    @pl.when(c == 0)
    def _():
        pltpu.sync_copy(s_sh.at[pl.ds(s * RPW, RPW)],
                        s0_hbm.at[pl.ds(s * RPW, RPW)])
        pltpu.sync_copy(deg_sh.at[pl.ds(s * RPW, RPW)],
                        d0_hbm.at[pl.ds(s * RPW, RPW)])

    @pl.when(c == 1)
    def _():
        pltpu.sync_copy(s_sh.at[pl.ds(s * RPW, RPW)],
                        s1_hbm.at[pl.ds(s * RPW, RPW)])
        pltpu.sync_copy(deg_sh.at[pl.ds(s * RPW, RPW)],
                        d1_hbm.at[pl.ds(s * RPW, RPW)])


@functools.lru_cache(maxsize=1)
def _make_sc_segsum():
  return functools.partial(
    pl.kernel,
    out_type=(
        jax.ShapeDtypeStruct((N_PAD, D), jnp.float32),
        jax.ShapeDtypeStruct((N_PAD, D), jnp.float32),
        jax.ShapeDtypeStruct((N_PAD, LANES), jnp.float32),
        jax.ShapeDtypeStruct((N_PAD, LANES), jnp.float32),
    ),
    mesh=plsc.VectorSubcoreMesh(core_axis_name="c", subcore_axis_name="s",
                                num_cores=NC, num_subcores=NS),
    scratch_types=[
        pltpu.VMEM((IBLK, CHUNK), jnp.int32),
        pltpu.VMEM((IBLK, CHUNK), jnp.int32),
        pltpu.VMEM((CHUNK, D), jnp.float32),
        pltpu.VMEM((CHUNK, D), jnp.float32),
        pltpu.VMEM((CHUNK, LANES), jnp.float32),
        pltpu.VMEM((ZROWS, D), jnp.float32),
        pltpu.VMEM((16, LANES), jnp.float32),
        pltpu.VMEM_SHARED((N_PAD, D), jnp.float32),
        pltpu.VMEM_SHARED((N_PAD, LANES), jnp.float32),
        pltpu.SemaphoreType.DMA,
        pltpu.SemaphoreType.DMA,
        pltpu.SemaphoreType.DMA,
        pltpu.SemaphoreType.DMA,
        pltpu.SemaphoreType.DMA,
        pltpu.SemaphoreType.DMA,
    ],
    compiler_params=pltpu.CompilerParams(use_tc_tiling_on_sc=False),
  )(_sc_body)


def _tc_a_body(x_ref, we_ref, be_ref, wn_ref, bn_ref, z_ref, m1t_ref):
    w1 = we_ref[:, :D]
    w2 = we_ref[:, D:]
    wnt = wn_ref[...].T
    zx = jnp.dot(x_ref[...], w2.T, preferred_element_type=jnp.float32) + be_ref[...]
    z_ref[...] = jnp.dot(zx, wnt, preferred_element_type=jnp.float32) + bn_ref[...]
    m1t_ref[...] = jnp.dot(w1.T, wnt, preferred_element_type=jnp.float32)


def _tc_a(x, w_edge, b_edge, w_node, b_node):
    blk = 1000
    row_spec = pl.BlockSpec((blk, D), lambda i: (i, 0))
    full = lambda a, b: pl.BlockSpec((a, b), lambda i: (0, 0))
    return pl.pallas_call(
        _tc_a_body,
        grid=(N // blk,),
        in_specs=[row_spec, full(D, 2 * D), full(1, D), full(D, D), full(1, D)],
        out_specs=[row_spec, full(D, D)],
        out_shape=[
            jax.ShapeDtypeStruct((N, D), jnp.float32),
            jax.ShapeDtypeStruct((D, D), jnp.float32),
        ],
    )(x, w_edge, b_edge, w_node, b_node)


def _tc_b_body(x_ref, s0_hbm, s1_hbm, d0_hbm, d1_hbm, z_ref, m1t_ref,
               out_ref, dv0, dv1, sb0, sb1, semd0, semd1, sem0, sem1):
    blk = out_ref.shape[0]
    i = pl.program_id(0)
    nblk = pl.num_programs(0)

    def s_copy(blk_idx, slot):
        c0 = pltpu.make_async_copy(
            s0_hbm.at[pl.ds(blk_idx * blk, blk)], sb0.at[slot], sem0)
        c1 = pltpu.make_async_copy(
            s1_hbm.at[pl.ds(blk_idx * blk, blk)], sb1.at[slot], sem1)
        return c0, c1

    @pl.when(i == 0)
    def _():
        cd0 = pltpu.make_async_copy(d0_hbm, dv0, semd0)
        cd1 = pltpu.make_async_copy(d1_hbm, dv1, semd1)
        cd0.start()
        cd1.start()
        c0, c1 = s_copy(0, 0)
        c0.start()
        c1.start()
        cd0.wait()
        cd1.wait()

    cw0, cw1 = s_copy(i, i % 2)
    cw0.wait()
    cw1.wait()

    @pl.when(i + 1 < nblk)
    def _():
        c0, c1 = s_copy(i + 1, (i + 1) % 2)
        c0.start()
        c1.start()

    deg = (dv0[pl.ds(i * blk, blk), 0:1] + dv1[pl.ds(i * blk, blk), 0:1])
    inv = 1.0 / jnp.maximum(deg, 1.0)
    mean_s = (sb0[i % 2] + sb1[i % 2]) * inv
    h = jnp.dot(mean_s, m1t_ref[...], preferred_element_type=jnp.float32) + z_ref[...]
    out_ref[...] = jnp.where(deg > 0.0, h, x_ref[...])


def _tc_b(x, s0, s1, d0, d1, z, m1t):
    blk = 1000
    row_spec = pl.BlockSpec((blk, D), lambda i: (i, 0))
    hbm_spec = pl.BlockSpec(memory_space=pltpu.MemorySpace.HBM)
    full = lambda a, b: pl.BlockSpec((a, b), lambda i: (0, 0))
    return pl.pallas_call(
        _tc_b_body,
        grid=(N // blk,),
        in_specs=[row_spec, hbm_spec, hbm_spec, hbm_spec, hbm_spec,
                  row_spec, full(D, D)],
        out_specs=row_spec,
        out_shape=jax.ShapeDtypeStruct((N, D), jnp.float32),
        scratch_shapes=[
            pltpu.VMEM((N_PAD, LANES), jnp.float32),
            pltpu.VMEM((N_PAD, LANES), jnp.float32),
            pltpu.VMEM((2, blk, D), jnp.float32),
            pltpu.VMEM((2, blk, D), jnp.float32),
            pltpu.SemaphoreType.DMA,
            pltpu.SemaphoreType.DMA,
            pltpu.SemaphoreType.DMA,
            pltpu.SemaphoreType.DMA,
        ],
    )(x, s0, s1, d0, d1, z, m1t)


def kernel(node_inputs, edge_index, W_edge, b_edge, W_node, b_node):
    npad = EPAD - E
    fsrc = jnp.arange(npad, dtype=jnp.int32) * 13 % N
    src_p = jnp.concatenate([edge_index[0], fsrc])
    fakes = N + jnp.arange(npad, dtype=jnp.int32) % (N_PAD - N)
    dst_p = jnp.concatenate([edge_index[1], fakes])
    ei = jnp.stack([src_p, dst_p]).reshape(2, NCHUNK, CHUNK)
    s0, s1, d0, d1 = _make_sc_segsum()(node_inputs, ei)
    z, m1t = _tc_a(node_inputs, W_edge, b_edge.reshape(1, D),
                   W_node, b_node.reshape(1, D))
    return _tc_b(node_inputs, s0, s1, d0, d1, z, m1t)


# speedup vs baseline: 3.2822x; 1.0784x over previous
import functools

import jax
import jax.numpy as jnp
from jax import lax
from jax.experimental import pallas as pl
from jax.experimental.pallas import tpu as pltpu
from jax.experimental.pallas import tpu_sc as plsc

N = 10000
E = 320000
D = 128
LANES = 16
NC = 2
NS = 16
NW = NC * NS
CHUNK = 128
NCHUNK = E // CHUNK
IBLK = 10
TBLK = NCHUNK // IBLK
FULLW = TBLK % NW
N_PAD = 10240
RPW = N_PAD // NS
ZROWS = 8


def _sc_body(x_hbm, ei_hbm, s0_hbm, s1_hbm, d0_hbm, d1_hbm,
             idx_s, idx_d, rows0, rows1, ones, zbuf, zdeg, s_sh, deg_sh,
             sem_g0, sem_g1, sem_s0, sem_s1, sem_o0, sem_o1):
    c = lax.axis_index("c")
    s = lax.axis_index("s")
    wid = c * NS + s
    rows = (rows0, rows1)
    sem_g = (sem_g0, sem_g1)
    sem_s = (sem_s0, sem_s1)
    sem_o = (sem_o0, sem_o1)

    def zbuf_body(i, _):
        zbuf[i // 8, pl.ds((i % 8) * LANES, LANES)] = jnp.zeros((LANES,), jnp.float32)
        return 0
    lax.fori_loop(0, ZROWS * 8, zbuf_body, 0)

    def zdeg_body(i, _):
        zdeg[i] = jnp.zeros((LANES,), jnp.float32)
        return 0
    lax.fori_loop(0, 16, zdeg_body, 0)

    def ones_body(i, _):
        ones[i] = jnp.ones((LANES,), jnp.float32)
        return 0
    lax.fori_loop(0, CHUNK, ones_body, 0)

    def z_s(k, _):
        pltpu.sync_copy(zbuf, s_sh.at[pl.ds(s * RPW + k * ZROWS, ZROWS)])
        return 0
    lax.fori_loop(0, RPW // ZROWS, z_s, 0)

    def z_d(k, _):
        pltpu.sync_copy(zdeg, deg_sh.at[pl.ds(s * RPW + k * 16, 16)])
        return 0
    lax.fori_loop(0, RPW // 16, z_d, 0)
    plsc.subcore_barrier()

    def block_body(b, _):
        base = (wid + b * NW) * IBLK
        pltpu.sync_copy(ei_hbm.at[0, pl.ds(base, IBLK)], idx_s)
        pltpu.sync_copy(ei_hbm.at[1, pl.ds(base, IBLK)], idx_d)
        g = [None, None]
        sc = [None, None]
        oc = [None, None]
        g[0] = pltpu.async_copy(x_hbm.at[idx_s.at[0]], rows0, sem_g0)
        for j in range(IBLK):
            p = j & 1
            q = 1 - p
            g[p].wait()
            if sc[q] is not None:
                sc[q].wait()
                oc[q].wait()
            if j + 1 < IBLK:
                g[q] = pltpu.async_copy(
                    x_hbm.at[idx_s.at[j + 1]], rows[q], sem_g[q])
            sc[p] = pltpu.async_copy(
                rows[p], s_sh.at[idx_d.at[j]], sem_s[p], add=True)
            oc[p] = pltpu.async_copy(
                ones, deg_sh.at[idx_d.at[j]], sem_o[p], add=True)
        last = (IBLK - 1) & 1
        sc[last].wait()
        oc[last].wait()
        return 0
    nb = jnp.where(wid < FULLW, TBLK // NW + 1, TBLK // NW)
    lax.fori_loop(0, nb, block_body, 0)
    plsc.subcore_barrier()

    @pl.when(c == 0)
    def _():
        pltpu.sync_copy(s_sh.at[pl.ds(s * RPW, RPW)],
                        s0_hbm.at[pl.ds(s * RPW, RPW)])
        pltpu.sync_copy(deg_sh.at[pl.ds(s * RPW, RPW)],
                        d0_hbm.at[pl.ds(s * RPW, RPW)])

    @pl.when(c == 1)
    def _():
        pltpu.sync_copy(s_sh.at[pl.ds(s * RPW, RPW)],
                        s1_hbm.at[pl.ds(s * RPW, RPW)])
        pltpu.sync_copy(deg_sh.at[pl.ds(s * RPW, RPW)],
                        d1_hbm.at[pl.ds(s * RPW, RPW)])


@functools.lru_cache(maxsize=1)
def _make_sc_segsum():
  return functools.partial(
    pl.kernel,
    out_type=(
        jax.ShapeDtypeStruct((N_PAD, D), jnp.float32),
        jax.ShapeDtypeStruct((N_PAD, D), jnp.float32),
        jax.ShapeDtypeStruct((N_PAD, LANES), jnp.float32),
        jax.ShapeDtypeStruct((N_PAD, LANES), jnp.float32),
    ),
    mesh=plsc.VectorSubcoreMesh(core_axis_name="c", subcore_axis_name="s",
                                num_cores=NC, num_subcores=NS),
    scratch_types=[
        pltpu.VMEM((IBLK, CHUNK), jnp.int32),
        pltpu.VMEM((IBLK, CHUNK), jnp.int32),
        pltpu.VMEM((CHUNK, D), jnp.float32),
        pltpu.VMEM((CHUNK, D), jnp.float32),
        pltpu.VMEM((CHUNK, LANES), jnp.float32),
        pltpu.VMEM((ZROWS, D), jnp.float32),
        pltpu.VMEM((16, LANES), jnp.float32),
        pltpu.VMEM_SHARED((N_PAD, D), jnp.float32),
        pltpu.VMEM_SHARED((N_PAD, LANES), jnp.float32),
        pltpu.SemaphoreType.DMA,
        pltpu.SemaphoreType.DMA,
        pltpu.SemaphoreType.DMA,
        pltpu.SemaphoreType.DMA,
        pltpu.SemaphoreType.DMA,
        pltpu.SemaphoreType.DMA,
    ],
    compiler_params=pltpu.CompilerParams(use_tc_tiling_on_sc=False),
  )(_sc_body)


def _tc_a_body(x_ref, we_ref, be_ref, wn_ref, bn_ref, z_ref, m1t_ref):
    w1 = we_ref[:, :D]
    w2 = we_ref[:, D:]
    wnt = wn_ref[...].T
    zx = jnp.dot(x_ref[...], w2.T, preferred_element_type=jnp.float32) + be_ref[...]
    z_ref[...] = jnp.dot(zx, wnt, preferred_element_type=jnp.float32) + bn_ref[...]
    m1t_ref[...] = jnp.dot(w1.T, wnt, preferred_element_type=jnp.float32)


def _tc_a(x, w_edge, b_edge, w_node, b_node):
    blk = 1000
    row_spec = pl.BlockSpec((blk, D), lambda i: (i, 0))
    full = lambda a, b: pl.BlockSpec((a, b), lambda i: (0, 0))
    return pl.pallas_call(
        _tc_a_body,
        grid=(N // blk,),
        in_specs=[row_spec, full(D, 2 * D), full(1, D), full(D, D), full(1, D)],
        out_specs=[row_spec, full(D, D)],
        out_shape=[
            jax.ShapeDtypeStruct((N, D), jnp.float32),
            jax.ShapeDtypeStruct((D, D), jnp.float32),
        ],
    )(x, w_edge, b_edge, w_node, b_node)


def _tc_b_body(x_ref, s0_ref, s1_ref, ds_ref, z_ref, m1t_ref, out_ref):
    deg = ds_ref[:, 0:1]
    inv = 1.0 / jnp.maximum(deg, 1.0)
    mean_s = (s0_ref[...] + s1_ref[...]) * inv
    h = jnp.dot(mean_s, m1t_ref[...], preferred_element_type=jnp.float32) + z_ref[...]
    out_ref[...] = jnp.where(deg > 0.0, h, x_ref[...])


def _tc_b(x, s0, s1, dsum, z, m1t):
    blk = 1000
    row_spec = pl.BlockSpec((blk, D), lambda i: (i, 0))
    deg_spec = pl.BlockSpec((blk, LANES), lambda i: (i, 0))
    full = lambda a, b: pl.BlockSpec((a, b), lambda i: (0, 0))
    return pl.pallas_call(
        _tc_b_body,
        grid=(N // blk,),
        in_specs=[row_spec, row_spec, row_spec, deg_spec, row_spec, full(D, D)],
        out_specs=row_spec,
        out_shape=jax.ShapeDtypeStruct((N, D), jnp.float32),
    )(x, s0, s1, dsum, z, m1t)


def kernel(node_inputs, edge_index, W_edge, b_edge, W_node, b_node):
    ei = edge_index.reshape(2, NCHUNK, CHUNK)
    s0, s1, d0, d1 = _make_sc_segsum()(node_inputs, ei)
    z, m1t = _tc_a(node_inputs, W_edge, b_edge.reshape(1, D),
                   W_node, b_node.reshape(1, D))
    return _tc_b(node_inputs, s0, s1, d0 + d1, z, m1t)


# speedup vs baseline: 3.3335x; 1.0156x over previous
import functools

import jax
import jax.numpy as jnp
from jax import lax
from jax.experimental import pallas as pl
from jax.experimental.pallas import tpu as pltpu
from jax.experimental.pallas import tpu_sc as plsc

N = 10000
E = 320000
D = 128
LANES = 16
NC = 2
NS = 16
NW = NC * NS
CHUNK = 128
NCHUNK = E // CHUNK
IBLK = 10
TBLK = NCHUNK // IBLK
FULLW = TBLK % NW
N_PAD = 10240
RPW = N_PAD // NS
ZROWS = 8


def _sc_body(x_hbm, ei_hbm, s0_hbm, s1_hbm, d0_hbm, d1_hbm,
             idx_s, idx_d, rows0, rows1, ones, zbuf, zdeg, s_sh, deg_sh,
             sem_g0, sem_g1, sem_s0, sem_s1, sem_o0, sem_o1):
    c = lax.axis_index("c")
    s = lax.axis_index("s")
    wid = c * NS + s
    rows = (rows0, rows1)
    sem_g = (sem_g0, sem_g1)
    sem_s = (sem_s0, sem_s1)
    sem_o = (sem_o0, sem_o1)

    def zbuf_body(i, _):
        zbuf[i // 8, pl.ds((i % 8) * LANES, LANES)] = jnp.zeros((LANES,), jnp.float32)
        return 0
    lax.fori_loop(0, ZROWS * 8, zbuf_body, 0)

    def zdeg_body(i, _):
        zdeg[i] = jnp.zeros((LANES,), jnp.float32)
        return 0
    lax.fori_loop(0, 16, zdeg_body, 0)

    def ones_body(i, _):
        ones[i] = jnp.ones((LANES,), jnp.float32)
        return 0
    lax.fori_loop(0, CHUNK, ones_body, 0)

    def z_s(k, _):
        pltpu.sync_copy(zbuf, s_sh.at[pl.ds(s * RPW + k * ZROWS, ZROWS)])
        return 0
    lax.fori_loop(0, RPW // ZROWS, z_s, 0)

    def z_d(k, _):
        pltpu.sync_copy(zdeg, deg_sh.at[pl.ds(s * RPW + k * 16, 16)])
        return 0
    lax.fori_loop(0, RPW // 16, z_d, 0)
    plsc.subcore_barrier()

    def block_body(b, _):
        base = (wid + b * NW) * IBLK * CHUNK
        pltpu.sync_copy(ei_hbm.at[0, pl.ds(base, IBLK * CHUNK)], idx_s)
        pltpu.sync_copy(ei_hbm.at[1, pl.ds(base, IBLK * CHUNK)], idx_d)
        g = [None, None]
        sc = [None, None]
        oc = [None, None]
        g[0] = pltpu.async_copy(
            x_hbm.at[idx_s.at[pl.ds(0, CHUNK)]], rows0, sem_g0)
        for j in range(IBLK):
            p = j & 1
            q = 1 - p
            g[p].wait()
            if sc[q] is not None:
                sc[q].wait()
                oc[q].wait()
            if j + 1 < IBLK:
                g[q] = pltpu.async_copy(
                    x_hbm.at[idx_s.at[pl.ds((j + 1) * CHUNK, CHUNK)]],
                    rows[q], sem_g[q])
            sc[p] = pltpu.async_copy(
                rows[p], s_sh.at[idx_d.at[pl.ds(j * CHUNK, CHUNK)]],
                sem_s[p], add=True)
            oc[p] = pltpu.async_copy(
                ones, deg_sh.at[idx_d.at[pl.ds(j * CHUNK, CHUNK)]],
                sem_o[p], add=True)
        last = (IBLK - 1) & 1
        sc[last].wait()
        oc[last].wait()
        return 0
    nb = jnp.where(wid < FULLW, TBLK // NW + 1, TBLK // NW)
    lax.fori_loop(0, nb, block_body, 0)
    plsc.subcore_barrier()

    @pl.when(c == 0)
    def _():
        pltpu.sync_copy(s_sh.at[pl.ds(s * RPW, RPW)],
                        s0_hbm.at[pl.ds(s * RPW, RPW)])
        pltpu.sync_copy(deg_sh.at[pl.ds(s * RPW, RPW)],
                        d0_hbm.at[pl.ds(s * RPW, RPW)])

    @pl.when(c == 1)
    def _():
        pltpu.sync_copy(s_sh.at[pl.ds(s * RPW, RPW)],
                        s1_hbm.at[pl.ds(s * RPW, RPW)])
        pltpu.sync_copy(deg_sh.at[pl.ds(s * RPW, RPW)],
                        d1_hbm.at[pl.ds(s * RPW, RPW)])


@functools.lru_cache(maxsize=1)
def _make_sc_segsum():
  return functools.partial(
    pl.kernel,
    out_type=(
        jax.ShapeDtypeStruct((N_PAD, D), jnp.float32),
        jax.ShapeDtypeStruct((N_PAD, D), jnp.float32),
        jax.ShapeDtypeStruct((N_PAD, LANES), jnp.float32),
        jax.ShapeDtypeStruct((N_PAD, LANES), jnp.float32),
    ),
    mesh=plsc.VectorSubcoreMesh(core_axis_name="c", subcore_axis_name="s",
                                num_cores=NC, num_subcores=NS),
    scratch_types=[
        pltpu.VMEM((IBLK * CHUNK,), jnp.int32),
        pltpu.VMEM((IBLK * CHUNK,), jnp.int32),
        pltpu.VMEM((CHUNK, D), jnp.float32),
        pltpu.VMEM((CHUNK, D), jnp.float32),
        pltpu.VMEM((CHUNK, LANES), jnp.float32),
        pltpu.VMEM((ZROWS, D), jnp.float32),
        pltpu.VMEM((16, LANES), jnp.float32),
        pltpu.VMEM_SHARED((N_PAD, D), jnp.float32),
        pltpu.VMEM_SHARED((N_PAD, LANES), jnp.float32),
        pltpu.SemaphoreType.DMA,
        pltpu.SemaphoreType.DMA,
        pltpu.SemaphoreType.DMA,
        pltpu.SemaphoreType.DMA,
        pltpu.SemaphoreType.DMA,
        pltpu.SemaphoreType.DMA,
    ],
    compiler_params=pltpu.CompilerParams(use_tc_tiling_on_sc=False),
  )(_sc_body)


def _tc_a_body(x_ref, we_ref, be_ref, wn_ref, bn_ref, z_ref, m1t_ref):
    w1 = we_ref[:, :D]
    w2 = we_ref[:, D:]
    wnt = wn_ref[...].T
    zx = jnp.dot(x_ref[...], w2.T, preferred_element_type=jnp.float32) + be_ref[...]
    z_ref[...] = jnp.dot(zx, wnt, preferred_element_type=jnp.float32) + bn_ref[...]
    m1t_ref[...] = jnp.dot(w1.T, wnt, preferred_element_type=jnp.float32)


def _tc_a(x, w_edge, b_edge, w_node, b_node):
    blk = 1000
    row_spec = pl.BlockSpec((blk, D), lambda i: (i, 0))
    full = lambda a, b: pl.BlockSpec((a, b), lambda i: (0, 0))
    return pl.pallas_call(
        _tc_a_body,
        grid=(N // blk,),
        in_specs=[row_spec, full(D, 2 * D), full(1, D), full(D, D), full(1, D)],
        out_specs=[row_spec, full(D, D)],
        out_shape=[
            jax.ShapeDtypeStruct((N, D), jnp.float32),
            jax.ShapeDtypeStruct((D, D), jnp.float32),
        ],
    )(x, w_edge, b_edge, w_node, b_node)


def _tc_b_body(x_ref, s0_ref, s1_ref, d0_ref, d1_ref, z_ref, m1t_ref, out_ref):
    deg = d0_ref[:, 0:1] + d1_ref[:, 0:1]
    inv = 1.0 / jnp.maximum(deg, 1.0)
    mean_s = (s0_ref[...] + s1_ref[...]) * inv
    h = jnp.dot(mean_s, m1t_ref[...], preferred_element_type=jnp.float32) + z_ref[...]
    out_ref[...] = jnp.where(deg > 0.0, h, x_ref[...])


def _tc_b(x, s0, s1, d0, d1, z, m1t):
    blk = 1000
    row_spec = pl.BlockSpec((blk, D), lambda i: (i, 0))
    deg_spec = pl.BlockSpec((blk, LANES), lambda i: (i, 0))
    full = lambda a, b: pl.BlockSpec((a, b), lambda i: (0, 0))
    return pl.pallas_call(
        _tc_b_body,
        grid=(N // blk,),
        in_specs=[row_spec, row_spec, row_spec, deg_spec, deg_spec,
                  row_spec, full(D, D)],
        out_specs=row_spec,
        out_shape=jax.ShapeDtypeStruct((N, D), jnp.float32),
    )(x, s0, s1, d0, d1, z, m1t)


def kernel(node_inputs, edge_index, W_edge, b_edge, W_node, b_node):
    s0, s1, d0, d1 = _make_sc_segsum()(node_inputs, edge_index)
    z, m1t = _tc_a(node_inputs, W_edge, b_edge.reshape(1, D),
                   W_node, b_node.reshape(1, D))
    return _tc_b(node_inputs, s0, s1, d0, d1, z, m1t)
